# baseline (speedup 1.0000x reference)
import functools

import jax
import jax.numpy as jnp
from jax import lax
from jax.experimental import pallas as pl
from jax.experimental.pallas import tpu as pltpu
from jax.experimental.pallas import tpu_sc as plsc

N, C, H, W = 4, 96, 384, 384
P = H * W
NP = N * P
NW = 32
PPW = NP // NW
CH = 128
CHUNKS = PPW // CH
G16 = CH // 16


def _sc_body(table_hbm, gx_hbm, gy_hbm, out_hbm,
             gx_v, gy_v,
             i00_v, i01_v, i10_v, i11_v,
             w00_v, w01_v, w10_v, w11_v,
             r00_v, r01_v, r10_v, r11_v,
             out_v, sem_g):
    cid = lax.axis_index("c")
    sid = lax.axis_index("s")
    wid = sid * 2 + cid
    base = wid * PPW
    nbase = (base // P) * P

    def chunk_body(k, carry):
        off = base + k * CH
        pltpu.sync_copy(gx_hbm.at[pl.ds(off, CH)], gx_v)
        pltpu.sync_copy(gy_hbm.at[pl.ds(off, CH)], gy_v)

        def idx_body(g, c2):
            s = pl.ds(g * 16, 16)
            x = gx_v[s]
            y = gy_v[s]
            ix = ((x + 1.0) * W - 1.0) * 0.5
            iy = ((y + 1.0) * H - 1.0) * 0.5
            ixt = ix.astype(jnp.int32)
            ixtf = ixt.astype(jnp.float32)
            mx = ix < ixtf
            ix0 = ixt - jnp.where(mx, 1, 0)
            fx0 = ixtf - jnp.where(mx, 1.0, 0.0)
            iyt = iy.astype(jnp.int32)
            iytf = iyt.astype(jnp.float32)
            my = iy < iytf
            iy0 = iyt - jnp.where(my, 1, 0)
            fy0 = iytf - jnp.where(my, 1.0, 0.0)
            wx1 = ix - fx0
            wx0 = 1.0 - wx1
            wy1 = iy - fy0
            wy0 = 1.0 - wy1
            vx0 = (ix0 >= 0) & (ix0 <= W - 1)
            vx1 = (ix0 >= -1) & (ix0 <= W - 2)
            vy0 = (iy0 >= 0) & (iy0 <= H - 1)
            vy1 = (iy0 >= -1) & (iy0 <= H - 2)
            wx0 = jnp.where(vx0, wx0, 0.0)
            wx1 = jnp.where(vx1, wx1, 0.0)
            wy0 = jnp.where(vy0, wy0, 0.0)
            wy1 = jnp.where(vy1, wy1, 0.0)
            cx0 = jnp.minimum(jnp.maximum(ix0, 0), W - 1)
            cx1 = jnp.minimum(jnp.maximum(ix0 + 1, 0), W - 1)
            cy0 = jnp.minimum(jnp.maximum(iy0, 0), H - 1)
            cy1 = jnp.minimum(jnp.maximum(iy0 + 1, 0), H - 1)
            rb0 = cy0 * W + nbase
            rb1 = cy1 * W + nbase
            i00_v[s] = rb0 + cx0
            i01_v[s] = rb0 + cx1
            i10_v[s] = rb1 + cx0
            i11_v[s] = rb1 + cx1
            w00_v[s] = wy0 * wx0
            w01_v[s] = wy0 * wx1
            w10_v[s] = wy1 * wx0
            w11_v[s] = wy1 * wx1
            return c2

        lax.fori_loop(0, G16, idx_body, 0)

        cp0 = pltpu.make_async_copy(table_hbm.at[i00_v], r00_v, sem_g)
        cp1 = pltpu.make_async_copy(table_hbm.at[i01_v], r01_v, sem_g)
        cp2 = pltpu.make_async_copy(table_hbm.at[i10_v], r10_v, sem_g)
        cp3 = pltpu.make_async_copy(table_hbm.at[i11_v], r11_v, sem_g)
        cp0.start()
        cp1.start()
        cp2.start()
        cp3.start()
        cp0.wait()
        cp1.wait()
        cp2.wait()
        cp3.wait()

        def cmb_body(g, c2):
            s = pl.ds(g * 16, 16)
            w00g = w00_v[s]
            w01g = w01_v[s]
            w10g = w10_v[s]
            w11g = w11_v[s]
            p0 = g * 16
            for i in range(16):
                px = p0 + i
                for j in range(C // 16):
                    cs = pl.ds(j * 16, 16)
                    acc = (r00_v[px, cs] * w00g[i]
                           + r01_v[px, cs] * w01g[i]
                           + r10_v[px, cs] * w10g[i]
                           + r11_v[px, cs] * w11g[i])
                    out_v[px, cs] = acc
            return c2

        lax.fori_loop(0, G16, cmb_body, 0)

        pltpu.sync_copy(out_v, out_hbm.at[pl.ds(off, CH)])
        return carry

    lax.fori_loop(0, CHUNKS, chunk_body, 0)


@jax.jit
def kernel(input, grid):
    table = jnp.transpose(input, (0, 2, 3, 1)).reshape(NP, C)
    gx = grid[..., 0].reshape(NP)
    gy = grid[..., 1].reshape(NP)

    mesh = plsc.VectorSubcoreMesh(core_axis_name="c", subcore_axis_name="s")
    out_rows = pl.kernel(
        _sc_body,
        out_type=jax.ShapeDtypeStruct((NP, C), jnp.float32),
        mesh=mesh,
        scratch_types=[
            pltpu.VMEM((CH,), jnp.float32),
            pltpu.VMEM((CH,), jnp.float32),
            pltpu.VMEM((CH,), jnp.int32),
            pltpu.VMEM((CH,), jnp.int32),
            pltpu.VMEM((CH,), jnp.int32),
            pltpu.VMEM((CH,), jnp.int32),
            pltpu.VMEM((CH,), jnp.float32),
            pltpu.VMEM((CH,), jnp.float32),
            pltpu.VMEM((CH,), jnp.float32),
            pltpu.VMEM((CH,), jnp.float32),
            pltpu.VMEM((CH, C), jnp.float32),
            pltpu.VMEM((CH, C), jnp.float32),
            pltpu.VMEM((CH, C), jnp.float32),
            pltpu.VMEM((CH, C), jnp.float32),
            pltpu.VMEM((CH, C), jnp.float32),
            pltpu.SemaphoreType.DMA,
        ],
        compiler_params=pltpu.CompilerParams(use_tc_tiling_on_sc=False),
    )(table, gx, gy)

    return out_rows.reshape(N, H, W, C).transpose(0, 3, 1, 2)


# speedup vs baseline: 1.4044x; 1.4044x over previous
import jax
import jax.numpy as jnp
from jax import lax
from jax.experimental import pallas as pl
from jax.experimental.pallas import tpu as pltpu
from jax.experimental.pallas import tpu_sc as plsc

N, C, H, W = 4, 96, 384, 384
P = H * W
NP = N * P
NW = 32
PPW = NP // NW
CH = 96
CHUNKS = PPW // CH
G16 = CH // 16
CV = C // 16


def _sc_body(table_hbm, gx_hbm, gy_hbm, out_hbm,
             gx_v, gy_v, idx_v, w_v, r_v, out_v,
             sem_gr, sem_g, sem_o):
    cid = lax.axis_index("c")
    sid = lax.axis_index("s")
    wid = sid * 2 + cid
    base = wid * PPW
    nbase = (base // P) * P

    def start_grid(k, b):
        off = base + k * CH
        pltpu.make_async_copy(gx_hbm.at[pl.ds(off, CH)], gx_v.at[b], sem_gr.at[b]).start()
        pltpu.make_async_copy(gy_hbm.at[pl.ds(off, CH)], gy_v.at[b], sem_gr.at[b]).start()

    def wait_grid(k, b):
        off = base + k * CH
        pltpu.make_async_copy(gx_hbm.at[pl.ds(off, CH)], gx_v.at[b], sem_gr.at[b]).wait()
        pltpu.make_async_copy(gy_hbm.at[pl.ds(off, CH)], gy_v.at[b], sem_gr.at[b]).wait()

    def idx_compute(b):
        def idx_body(g, c2):
            s = pl.ds(g * 16, 16)
            x = gx_v[b, s]
            y = gy_v[b, s]
            ix = ((x + 1.0) * W - 1.0) * 0.5
            iy = ((y + 1.0) * H - 1.0) * 0.5
            ixt = ix.astype(jnp.int32)
            ixtf = ixt.astype(jnp.float32)
            mx = ix < ixtf
            ix0 = ixt - jnp.where(mx, 1, 0)
            fx0 = ixtf - jnp.where(mx, 1.0, 0.0)
            iyt = iy.astype(jnp.int32)
            iytf = iyt.astype(jnp.float32)
            my = iy < iytf
            iy0 = iyt - jnp.where(my, 1, 0)
            fy0 = iytf - jnp.where(my, 1.0, 0.0)
            wx1 = ix - fx0
            wx0 = 1.0 - wx1
            wy1 = iy - fy0
            wy0 = 1.0 - wy1
            vx0 = (ix0 >= 0) & (ix0 <= W - 1)
            vx1 = (ix0 >= -1) & (ix0 <= W - 2)
            vy0 = (iy0 >= 0) & (iy0 <= H - 1)
            vy1 = (iy0 >= -1) & (iy0 <= H - 2)
            wx0 = jnp.where(vx0, wx0, 0.0)
            wx1 = jnp.where(vx1, wx1, 0.0)
            wy0 = jnp.where(vy0, wy0, 0.0)
            wy1 = jnp.where(vy1, wy1, 0.0)
            cx0 = jnp.minimum(jnp.maximum(ix0, 0), W - 1)
            cx1 = jnp.minimum(jnp.maximum(ix0 + 1, 0), W - 1)
            cy0 = jnp.minimum(jnp.maximum(iy0, 0), H - 1)
            cy1 = jnp.minimum(jnp.maximum(iy0 + 1, 0), H - 1)
            rb0 = cy0 * W + nbase
            rb1 = cy1 * W + nbase
            idx_v[b, 0, s] = rb0 + cx0
            idx_v[b, 1, s] = rb0 + cx1
            idx_v[b, 2, s] = rb1 + cx0
            idx_v[b, 3, s] = rb1 + cx1
            w_v[b, 0, s] = wy0 * wx0
            w_v[b, 1, s] = wy0 * wx1
            w_v[b, 2, s] = wy1 * wx0
            w_v[b, 3, s] = wy1 * wx1
            return c2

        lax.fori_loop(0, G16, idx_body, 0)

    def start_gathers(b):
        for q in range(4):
            pltpu.make_async_copy(table_hbm.at[idx_v.at[b, q]], r_v.at[b, q],
                                  sem_g.at[b]).start()

    def wait_gathers(b):
        for q in range(4):
            pltpu.make_async_copy(table_hbm.at[idx_v.at[b, q]], r_v.at[b, q],
                                  sem_g.at[b]).wait()

    def combine(b):
        def cmb_body(g, c2):
            s = pl.ds(g * 16, 16)
            w00g = w_v[b, 0, s]
            w01g = w_v[b, 1, s]
            w10g = w_v[b, 2, s]
            w11g = w_v[b, 3, s]
            p0 = g * 16
            for i in range(16):
                px = p0 + i
                for j in range(CV):
                    cs = pl.ds(j * 16, 16)
                    acc = (r_v[b, 0, px, cs] * w00g[i]
                           + r_v[b, 1, px, cs] * w01g[i]
                           + r_v[b, 2, px, cs] * w10g[i]
                           + r_v[b, 3, px, cs] * w11g[i])
                    out_v[b, px, cs] = acc
            return c2

        lax.fori_loop(0, G16, cmb_body, 0)

    def start_out(k, b):
        off = base + k * CH
        pltpu.make_async_copy(out_v.at[b], out_hbm.at[pl.ds(off, CH)], sem_o.at[b]).start()

    def wait_out(k, b):
        off = base + k * CH
        pltpu.make_async_copy(out_v.at[b], out_hbm.at[pl.ds(off, CH)], sem_o.at[b]).wait()

    def step(k, b):
        def prefetch():
            wait_grid(k + 1, 1 - b)
            idx_compute(1 - b)
            start_gathers(1 - b)

        pl.when(k + 1 < CHUNKS)(prefetch)
        pl.when(k + 2 < CHUNKS)(lambda: start_grid(k + 2, b))
        wait_gathers(b)
        pl.when(k >= 2)(lambda: wait_out(k - 2, b))
        combine(b)
        start_out(k, b)

    start_grid(0, 0)
    wait_grid(0, 0)
    idx_compute(0)
    start_gathers(0)
    start_grid(1, 1)

    def loop_body(k2, carry):
        step(2 * k2, 0)
        step(2 * k2 + 1, 1)
        return carry

    lax.fori_loop(0, CHUNKS // 2, loop_body, 0)

    wait_out(CHUNKS - 2, 0)
    wait_out(CHUNKS - 1, 1)


@jax.jit
def kernel(input, grid):
    table = jnp.transpose(input, (0, 2, 3, 1)).reshape(NP, C)
    gx = grid[..., 0].reshape(NP)
    gy = grid[..., 1].reshape(NP)

    mesh = plsc.VectorSubcoreMesh(core_axis_name="c", subcore_axis_name="s")
    out_rows = pl.kernel(
        _sc_body,
        out_type=jax.ShapeDtypeStruct((NP, C), jnp.float32),
        mesh=mesh,
        scratch_types=[
            pltpu.VMEM((2, CH), jnp.float32),
            pltpu.VMEM((2, CH), jnp.float32),
            pltpu.VMEM((2, 4, CH), jnp.int32),
            pltpu.VMEM((2, 4, CH), jnp.float32),
            pltpu.VMEM((2, 4, CH, C), jnp.float32),
            pltpu.VMEM((2, CH, C), jnp.float32),
            pltpu.SemaphoreType.DMA((2,)),
            pltpu.SemaphoreType.DMA((2,)),
            pltpu.SemaphoreType.DMA((2,)),
        ],
        compiler_params=pltpu.CompilerParams(use_tc_tiling_on_sc=False),
    )(table, gx, gy)

    return out_rows.reshape(N, H, W, C).transpose(0, 3, 1, 2)
